# baseline (device time: 203912 ns/iter reference)
import jax
import jax.numpy as jnp
from jax import lax
from jax.experimental import pallas as pl
from jax.experimental.pallas import tpu as pltpu

N_DEV = 8
GENS = (1, 3, 4)


def kernel(A, B):
    m, _ = A.shape
    _, n = B.shape
    f32 = jnp.float32

    def body(a_ref, b_ref, out_ref, rbuf0, rbuf1, rbuf2,
             rs_send, rs_recv, ag_send, ag_recv):
        me = lax.axis_index("i")
        bit0 = me & 1
        bit1 = (me >> 1) & 1
        bit2 = (me >> 2) & 1
        c = (bit0 ^ bit1, bit1, bit2)

        barrier = pltpu.get_barrier_semaphore()
        for g in GENS:
            pl.semaphore_signal(
                barrier, inc=1,
                device_id=(me ^ g,), device_id_type=pl.DeviceIdType.MESH,
            )
        pl.semaphore_wait(barrier, 3)

        out_ref[:, :] = jnp.dot(a_ref[:, :], b_ref[:, :],
                                preferred_element_type=f32)

        rbufs = [rbuf0, rbuf1, rbuf2]

        start = jnp.int32(0)
        size = m
        for k in range(3):
            size //= 2
            ck = c[k]
            keep = start + ck * size
            send = start + (1 - ck) * size
            rdma = pltpu.make_async_remote_copy(
                src_ref=out_ref.at[pl.ds(send, size)],
                dst_ref=rbufs[k],
                send_sem=rs_send.at[k],
                recv_sem=rs_recv.at[k],
                device_id=(me ^ GENS[k],),
                device_id_type=pl.DeviceIdType.MESH,
            )
            rdma.start()
            rdma.wait()
            out_ref[pl.ds(keep, size), :] = (
                out_ref[pl.ds(keep, size), :] + rbufs[k][:, :]
            )
            start = keep

        z = out_ref[pl.ds(start, size), :]
        out_ref[pl.ds(start, size), :] = (
            0.5 * z * (1.0 + jnp.tanh(0.7978845608 * (z + 0.044715 * z * z * z)))
        )

        for k in (2, 1, 0):
            rdma = pltpu.make_async_remote_copy(
                src_ref=out_ref.at[pl.ds(start, size)],
                dst_ref=out_ref.at[pl.ds(start, size)],
                send_sem=ag_send.at[k],
                recv_sem=ag_recv.at[k],
                device_id=(me ^ GENS[k],),
                device_id_type=pl.DeviceIdType.MESH,
            )
            rdma.start()
            rdma.wait()
            start = start - c[k] * size
            size *= 2

    return pl.pallas_call(
        body,
        out_shape=jax.ShapeDtypeStruct((m, n), f32),
        in_specs=[
            pl.BlockSpec(memory_space=pltpu.VMEM),
            pl.BlockSpec(memory_space=pltpu.VMEM),
        ],
        out_specs=pl.BlockSpec(memory_space=pltpu.VMEM),
        scratch_shapes=[
            pltpu.VMEM((m // 2, n), f32),
            pltpu.VMEM((m // 4, n), f32),
            pltpu.VMEM((m // 8, n), f32),
            pltpu.SemaphoreType.DMA((3,)),
            pltpu.SemaphoreType.DMA((3,)),
            pltpu.SemaphoreType.DMA((3,)),
            pltpu.SemaphoreType.DMA((3,)),
        ],
        compiler_params=pltpu.CompilerParams(collective_id=0),
    )(A, B)


# device time: 87386 ns/iter; 2.3335x vs baseline; 2.3335x over previous
import jax
import jax.numpy as jnp
from jax import lax
from jax.experimental import pallas as pl
from jax.experimental.pallas import tpu as pltpu

N_DEV = 8
GENS = (1, 3, 4)


def kernel(A, B):
    m, _ = A.shape
    _, n = B.shape
    f32 = jnp.float32
    third = m // 3

    def body(a_ref, b_ref, out_ref, rb0, rb1, rb2,
             rs_send, rs_recv, ag_send, ag_recv):
        me = lax.axis_index("i")
        bit0 = me & 1
        bit1 = (me >> 1) & 1
        bit2 = (me >> 2) & 1
        c = (bit0 ^ bit1, bit1, bit2)

        barrier = pltpu.get_barrier_semaphore()
        for g in GENS:
            pl.semaphore_signal(
                barrier, inc=1,
                device_id=(me ^ g,), device_id_type=pl.DeviceIdType.MESH,
            )
        pl.semaphore_wait(barrier, 3)

        out_ref[:, :] = jnp.dot(a_ref[:, :], b_ref[:, :],
                                preferred_element_type=f32)

        rbufs = [rb0, rb1, rb2]

        starts = [jnp.int32(t * third) for t in range(3)]
        size = third
        for s in range(3):
            size //= 2
            rdmas = []
            keeps = []
            for t in range(3):
                a_idx = (t + s) % 3
                ck = c[a_idx]
                keep = starts[t] + ck * size
                send = starts[t] + (1 - ck) * size
                rdma = pltpu.make_async_remote_copy(
                    src_ref=out_ref.at[pl.ds(send, size)],
                    dst_ref=rbufs[s].at[t],
                    send_sem=rs_send.at[t, s],
                    recv_sem=rs_recv.at[t, s],
                    device_id=(me ^ GENS[a_idx],),
                    device_id_type=pl.DeviceIdType.MESH,
                )
                rdma.start()
                rdmas.append(rdma)
                keeps.append(keep)
            for t in range(3):
                rdmas[t].wait()
                out_ref[pl.ds(keeps[t], size), :] = (
                    out_ref[pl.ds(keeps[t], size), :] + rbufs[s][t]
                )
                starts[t] = keeps[t]

        for t in range(3):
            z = out_ref[pl.ds(starts[t], size), :]
            out_ref[pl.ds(starts[t], size), :] = (
                0.5 * z
                * (1.0 + jnp.tanh(0.7978845608 * (z + 0.044715 * z * z * z)))
            )

        for s in range(3):
            rdmas = []
            for t in range(3):
                a_idx = (t + 2 - s) % 3
                rdma = pltpu.make_async_remote_copy(
                    src_ref=out_ref.at[pl.ds(starts[t], size)],
                    dst_ref=out_ref.at[pl.ds(starts[t], size)],
                    send_sem=ag_send.at[t, s],
                    recv_sem=ag_recv.at[t, s],
                    device_id=(me ^ GENS[a_idx],),
                    device_id_type=pl.DeviceIdType.MESH,
                )
                rdma.start()
                rdmas.append(rdma)
            for t in range(3):
                rdmas[t].wait()
                a_idx = (t + 2 - s) % 3
                starts[t] = starts[t] - c[a_idx] * size
            size *= 2

    return pl.pallas_call(
        body,
        out_shape=jax.ShapeDtypeStruct((m, n), f32),
        in_specs=[
            pl.BlockSpec(memory_space=pltpu.VMEM),
            pl.BlockSpec(memory_space=pltpu.VMEM),
        ],
        out_specs=pl.BlockSpec(memory_space=pltpu.VMEM),
        scratch_shapes=[
            pltpu.VMEM((3, third // 2, n), f32),
            pltpu.VMEM((3, third // 4, n), f32),
            pltpu.VMEM((3, third // 8, n), f32),
            pltpu.SemaphoreType.DMA((3, 3)),
            pltpu.SemaphoreType.DMA((3, 3)),
            pltpu.SemaphoreType.DMA((3, 3)),
            pltpu.SemaphoreType.DMA((3, 3)),
        ],
        compiler_params=pltpu.CompilerParams(collective_id=0),
    )(A, B)


# device time: 56057 ns/iter; 3.6376x vs baseline; 1.5589x over previous
import jax
import jax.numpy as jnp
from jax import lax
from jax.experimental import pallas as pl
from jax.experimental.pallas import tpu as pltpu

N_DEV = 8
GENS = (1, 3, 4)


def kernel(A, B):
    m, _ = A.shape
    _, n = B.shape
    f32 = jnp.float32
    bf16 = jnp.bfloat16
    third = m // 3

    def body(a_ref, b_ref, out_ref, sb0, sb1, sb2, rb0, rb1, rb2, ag_buf,
             rs_send, rs_recv, ag_send, ag_recv):
        me = lax.axis_index("i")
        bit0 = me & 1
        bit1 = (me >> 1) & 1
        bit2 = (me >> 2) & 1
        c = (bit0 ^ bit1, bit1, bit2)

        barrier = pltpu.get_barrier_semaphore()
        for g in GENS:
            pl.semaphore_signal(
                barrier, inc=1,
                device_id=(me ^ g,), device_id_type=pl.DeviceIdType.MESH,
            )
        pl.semaphore_wait(barrier, 3)

        sbufs = [sb0, sb1, sb2]
        rbufs = [rb0, rb1, rb2]
        sizes = [third // 2, third // 4, third // 8]

        def mm(r0, rows):
            return jnp.dot(a_ref[pl.ds(r0, rows), :], b_ref[:, :],
                           preferred_element_type=f32)

        def make_rs(t, s):
            return pltpu.make_async_remote_copy(
                src_ref=sbufs[s].at[t],
                dst_ref=rbufs[s].at[t],
                send_sem=rs_send.at[t, s],
                recv_sem=rs_recv.at[t, s],
                device_id=(me ^ GENS[(t + s) % 3],),
                device_id_type=pl.DeviceIdType.MESH,
            )

        starts = []
        rdmas = [None, None, None]
        for t in range(3):
            ck = c[t]
            send = t * third + (1 - ck) * sizes[0]
            sbufs[0][t, :, :] = mm(send, sizes[0]).astype(bf16)
            rdmas[t] = make_rs(t, 0)
            rdmas[t].start()
            starts.append(t * third + ck * sizes[0])
        for t in range(3):
            out_ref[pl.ds(starts[t], sizes[0]), :] = mm(starts[t], sizes[0])

        for s in (1, 2):
            new_rdmas = [None, None, None]
            for t in range(3):
                rdmas[t].wait()
                acc = (out_ref[pl.ds(starts[t], sizes[s - 1]), :]
                       + rbufs[s - 1][t].astype(f32))
                ck = c[(t + s) % 3]
                keep = starts[t] + ck * sizes[s]
                send = starts[t] + (1 - ck) * sizes[s]
                out_ref[pl.ds(starts[t], sizes[s - 1]), :] = acc
                sbufs[s][t, :, :] = out_ref[pl.ds(send, sizes[s]), :].astype(bf16)
                new_rdmas[t] = make_rs(t, s)
                new_rdmas[t].start()
                starts[t] = keep
            rdmas = new_rdmas

        for t in range(3):
            rdmas[t].wait()
            z = (out_ref[pl.ds(starts[t], sizes[2]), :]
                 + rbufs[2][t].astype(f32))
            g = 0.5 * z * (1.0 + jnp.tanh(
                0.7978845608 * (z + 0.044715 * z * z * z)))
            ag_buf[pl.ds(starts[t], sizes[2]), :] = g.astype(bf16)

        size = sizes[2]
        for s in range(3):
            rdmas = [None, None, None]
            for t in range(3):
                rdmas[t] = pltpu.make_async_remote_copy(
                    src_ref=ag_buf.at[pl.ds(starts[t], size)],
                    dst_ref=ag_buf.at[pl.ds(starts[t], size)],
                    send_sem=ag_send.at[t, s],
                    recv_sem=ag_recv.at[t, s],
                    device_id=(me ^ GENS[(t + 2 - s) % 3],),
                    device_id_type=pl.DeviceIdType.MESH,
                )
                rdmas[t].start()
            for t in range(3):
                rdmas[t].wait()
                starts[t] = starts[t] - c[(t + 2 - s) % 3] * size
            size *= 2

        out_ref[:, :] = ag_buf[:, :].astype(f32)

    return pl.pallas_call(
        body,
        out_shape=jax.ShapeDtypeStruct((m, n), f32),
        in_specs=[
            pl.BlockSpec(memory_space=pltpu.VMEM),
            pl.BlockSpec(memory_space=pltpu.VMEM),
        ],
        out_specs=pl.BlockSpec(memory_space=pltpu.VMEM),
        scratch_shapes=[
            pltpu.VMEM((3, third // 2, n), bf16),
            pltpu.VMEM((3, third // 4, n), bf16),
            pltpu.VMEM((3, third // 8, n), bf16),
            pltpu.VMEM((3, third // 2, n), bf16),
            pltpu.VMEM((3, third // 4, n), bf16),
            pltpu.VMEM((3, third // 8, n), bf16),
            pltpu.VMEM((m, n), bf16),
            pltpu.SemaphoreType.DMA((3, 3)),
            pltpu.SemaphoreType.DMA((3, 3)),
            pltpu.SemaphoreType.DMA((3, 3)),
            pltpu.SemaphoreType.DMA((3, 3)),
        ],
        compiler_params=pltpu.CompilerParams(collective_id=0),
    )(A, B)
